# baseline (speedup 1.0000x reference)
import functools

import jax
import jax.numpy as jnp
from jax import lax
from jax.experimental import pallas as pl
from jax.experimental.pallas import tpu as pltpu
from jax.experimental.pallas import tpu_sc as plsc

N = 10000
E = 320000
D_IN = 128
H = 16
D_OUT = 128

NC = 2
NS = 16
NW = NC * NS
CHUNK = 128
CHUNKS_PW = 80
EPW = CHUNK * CHUNKS_PW
E_PAD = EPW * NW
N_ACC = 10240
ROWS_PT = N_ACC // NS
NBUF = 8
GAH = 4
N_Y = N // NS

NP = N // 8
NP_ACC = N_ACC // 8


def _sc_degree_body(row_hbm, ones_hbm, zeros_hbm, out_hbm,
                    rows_v, msg_v, acc_sh, sem):
    cid = lax.axis_index("c")
    sid = lax.axis_index("s")
    wid = cid * NS + sid
    pltpu.sync_copy(zeros_hbm.at[pl.ds(sid * ROWS_PT, ROWS_PT)],
                    acc_sh.at[pl.ds(sid * ROWS_PT, ROWS_PT)])
    pltpu.sync_copy(row_hbm.at[wid], rows_v)
    pltpu.sync_copy(ones_hbm, msg_v)
    plsc.subcore_barrier()

    def body(cb, carry):
        for j in range(8):
            pltpu.async_copy(msg_v, acc_sh.at[rows_v.at[cb * 8 + j]], sem,
                             add=True)
        for j in range(8):
            pltpu.make_async_copy(msg_v, acc_sh.at[rows_v.at[cb * 8 + j]],
                                  sem).wait()
        return carry
    lax.fori_loop(0, CHUNKS_PW // 8, body, 0)
    plsc.subcore_barrier()
    pltpu.sync_copy(acc_sh.at[pl.ds(sid * ROWS_PT, ROWS_PT)],
                    out_hbm.at[cid, pl.ds(sid * ROWS_PT, ROWS_PT)])


@functools.cache
def _sc_degree():
    mesh = plsc.VectorSubcoreMesh(
        core_axis_name="c", subcore_axis_name="s",
        num_cores=NC, num_subcores=NS)
    return pl.kernel(
        _sc_degree_body, mesh=mesh,
        out_type=jax.ShapeDtypeStruct((NC, N_ACC, H), jnp.float32),
        scratch_types=[
            pltpu.VMEM((CHUNKS_PW, CHUNK), jnp.int32),
            pltpu.VMEM((CHUNK, H), jnp.float32),
            pltpu.VMEM_SHARED((N_ACC, H), jnp.float32),
            pltpu.SemaphoreType.DMA,
        ],
        compiler_params=pltpu.CompilerParams(use_tc_tiling_on_sc=False),
        name="sc_degree",
    )


def _agg_loop(rows_v, cols_v, msg_v, acc_sh, y_sh, gsems, ssems):
    def slot(c, j, drain, gather):
        pltpu.make_async_copy(y_sh.at[cols_v.at[c]],
                              msg_v.at[j], gsems[j]).wait()
        pltpu.async_copy(msg_v.at[j], acc_sh.at[rows_v.at[c]], ssems[j],
                         add=True)
        if gather:
            bg = (j + GAH) % NBUF
            cg = c + GAH
            if drain:
                pltpu.make_async_copy(msg_v.at[bg],
                                      acc_sh.at[rows_v.at[cg - NBUF]],
                                      ssems[bg]).wait()
            pltpu.async_copy(y_sh.at[cols_v.at[cg]], msg_v.at[bg], gsems[bg])

    for b in range(GAH):
        pltpu.async_copy(y_sh.at[cols_v.at[b]], msg_v.at[b], gsems[b])
    for j in range(NBUF):
        slot(j, j, drain=(j >= GAH), gather=True)

    def blk(cb, carry):
        for j in range(NBUF):
            slot(cb * NBUF + j, j, drain=True, gather=True)
        return carry
    lax.fori_loop(1, CHUNKS_PW // NBUF - 1, blk, 0)

    last = CHUNKS_PW - NBUF
    for j in range(NBUF):
        slot(last + j, j, drain=(j < GAH), gather=(j < GAH))
    for j in range(NBUF):
        pltpu.make_async_copy(msg_v.at[j], acc_sh.at[rows_v.at[last + j]],
                              ssems[j]).wait()


def _sc_agg_body(row_hbm, col_hbm, y_hbm, zeros_hbm, out_hbm,
                 rows_v, cols_v, msg_v, acc_sh, y_sh, *sems):
    gsems, ssems = sems[:NBUF], sems[NBUF:]
    cid = lax.axis_index("c")
    sid = lax.axis_index("s")
    wid = cid * NS + sid
    pltpu.sync_copy(zeros_hbm.at[pl.ds(sid * ROWS_PT, ROWS_PT)],
                    acc_sh.at[pl.ds(sid * ROWS_PT, ROWS_PT)])
    pltpu.sync_copy(y_hbm.at[pl.ds(sid * N_Y, N_Y)],
                    y_sh.at[pl.ds(sid * N_Y, N_Y)])
    pltpu.sync_copy(row_hbm.at[wid], rows_v)
    pltpu.sync_copy(col_hbm.at[wid], cols_v)
    plsc.subcore_barrier()
    _agg_loop(rows_v, cols_v, msg_v, acc_sh, y_sh, gsems, ssems)
    plsc.subcore_barrier()
    pltpu.sync_copy(acc_sh.at[pl.ds(sid * ROWS_PT, ROWS_PT)],
                    out_hbm.at[cid, pl.ds(sid * ROWS_PT, ROWS_PT)])


def _sc_agg_scaled_body(row_hbm, col_hbm, z_hbm, dinv_hbm, zeros_hbm, out_hbm,
                        rows_v, cols_v, msg_v, acc_sh, y_sh, zb_v, db_v,
                        *sems):
    gsems, ssems = sems[:NBUF], sems[NBUF:]
    cid = lax.axis_index("c")
    sid = lax.axis_index("s")
    wid = cid * NS + sid
    pltpu.sync_copy(zeros_hbm.at[pl.ds(sid * ROWS_PT, ROWS_PT)],
                    acc_sh.at[pl.ds(sid * ROWS_PT, ROWS_PT)])
    pltpu.sync_copy(z_hbm.at[pl.ds(sid * N_Y, N_Y)], zb_v)
    pltpu.sync_copy(dinv_hbm.at[pl.ds(sid * N_Y, N_Y)], db_v)

    def scale(r, carry):
        zb_v[r] = zb_v[r] * db_v[r]
        return carry
    lax.fori_loop(0, N_Y, scale, 0)
    pltpu.sync_copy(zb_v, y_sh.at[pl.ds(sid * N_Y, N_Y)])
    pltpu.sync_copy(row_hbm.at[wid], rows_v)
    pltpu.sync_copy(col_hbm.at[wid], cols_v)
    plsc.subcore_barrier()
    _agg_loop(rows_v, cols_v, msg_v, acc_sh, y_sh, gsems, ssems)
    plsc.subcore_barrier()
    pltpu.sync_copy(acc_sh.at[pl.ds(sid * ROWS_PT, ROWS_PT)],
                    out_hbm.at[cid, pl.ds(sid * ROWS_PT, ROWS_PT)])


_AGG_SCRATCH = [
    pltpu.VMEM((CHUNKS_PW, CHUNK), jnp.int32),
    pltpu.VMEM((CHUNKS_PW, CHUNK), jnp.int32),
    pltpu.VMEM((NBUF, CHUNK, H), jnp.float32),
    pltpu.VMEM_SHARED((N_ACC, H), jnp.float32),
    pltpu.VMEM_SHARED((N, H), jnp.float32),
]
_SEMS = [pltpu.SemaphoreType.DMA] * (2 * NBUF)


@functools.cache
def _sc_agg():
    mesh = plsc.VectorSubcoreMesh(
        core_axis_name="c", subcore_axis_name="s",
        num_cores=NC, num_subcores=NS)
    return pl.kernel(
        _sc_agg_body, mesh=mesh,
        out_type=jax.ShapeDtypeStruct((NC, N_ACC, H), jnp.float32),
        scratch_types=_AGG_SCRATCH + _SEMS,
        compiler_params=pltpu.CompilerParams(use_tc_tiling_on_sc=False),
        name="sc_agg",
    )


@functools.cache
def _sc_agg_scaled():
    mesh = plsc.VectorSubcoreMesh(
        core_axis_name="c", subcore_axis_name="s",
        num_cores=NC, num_subcores=NS)
    return pl.kernel(
        _sc_agg_scaled_body, mesh=mesh,
        out_type=jax.ShapeDtypeStruct((NC, N_ACC, H), jnp.float32),
        scratch_types=_AGG_SCRATCH + [
            pltpu.VMEM((N_Y, H), jnp.float32),
            pltpu.VMEM((N_Y, H), jnp.float32),
        ] + _SEMS,
        compiler_params=pltpu.CompilerParams(use_tc_tiling_on_sc=False),
        name="sc_agg_scaled",
    )


def _fold8(v128):
    acc = v128[:, 0:H]
    for k in range(1, 8):
        acc = acc + v128[:, k * H:(k + 1) * H]
    return acc


def _tile8(v16):
    return jnp.concatenate([v16] * 8, axis=1)


def _bd_dot(hp, w):
    k = w.shape[0]
    return jnp.concatenate(
        [jnp.dot(hp[:, j * k:(j + 1) * k], w,
                 preferred_element_type=jnp.float32) for j in range(8)],
        axis=1)


def _bn_relu_packed(h, g, beta):
    mu = _tile8(_fold8(jnp.mean(h, axis=0, keepdims=True)) / 8.0)
    ex2 = _tile8(_fold8(jnp.mean(h * h, axis=0, keepdims=True)) / 8.0)
    var = ex2 - mu * mu
    return jnp.maximum((h - mu) / jnp.sqrt(var + 1e-5) * g + beta, 0.0)


def _tc1_body(degp, x, w1t, w1b, dinvp_o, zn_o, p1n_o):
    deg = degp[0, :NP] + degp[1, :NP] + 1.0
    dinvp_o[...] = lax.rsqrt(deg)
    zn_o[...] = jnp.dot(x[...], w1b[...], preferred_element_type=jnp.float32)
    p1n_o[...] = jnp.dot(x[...], w1t[...], preferred_element_type=jnp.float32)


def _tc2_body(s1p, dinvp, zp, p1p, b1, g1, bt1, w2t, v2p_o, p2p_o):
    dinv = dinvp[...]
    u1 = dinv * zp[...]
    h = p1p[...] + dinv * (s1p[0, :NP] + s1p[1, :NP] + u1) + _tile8(b1[...])
    hn = _bn_relu_packed(h, _tile8(g1[...]), _tile8(bt1[...]))
    v2p_o[...] = dinv * hn
    p2p_o[...] = _bd_dot(hn, w2t[...])


def _tc3_body(t2p, dinvp, v2p, p2p, b2, g2, bt2, w2b, w3t,
              v3p_o, p3p_o):
    dinv = dinvp[...]
    agg2 = dinv * (t2p[0, :NP] + t2p[1, :NP] + v2p[...])
    h = p2p[...] + _bd_dot(agg2, w2b[...]) + _tile8(b2[...])
    hn = _bn_relu_packed(h, _tile8(g2[...]), _tile8(bt2[...]))
    v3p_o[...] = dinv * hn
    p3p_o[...] = _bd_dot(hn, w3t[...])


def _tc4_body(t3p, dinvp, v3p, p3p, w3b, b3, outp_o):
    agg3 = dinvp[...] * (t3p[0, :NP] + t3p[1, :NP] + v3p[...])
    b3t = jnp.concatenate([b3[...]] * 8, axis=1)
    outp_o[...] = p3p[...] + _bd_dot(agg3, w3b[...]) + b3t


def kernel(edge_index, features, W1, b1, g1, beta1, W2, b2, g2, beta2, W3, b3):
    f32 = jnp.float32
    row = edge_index[0]
    col = edge_index[1]
    pad = E_PAD - E
    row_p = jnp.concatenate([row, jnp.full((pad,), N, jnp.int32)])
    col_p = jnp.concatenate([col, jnp.zeros((pad,), jnp.int32)])
    row_p = row_p.reshape(NW, CHUNKS_PW, CHUNK)
    col_p = col_p.reshape(NW, CHUNKS_PW, CHUNK)
    zeros_acc = jnp.zeros((N_ACC, H), f32)
    ones_msg = jnp.ones((CHUNK, H), f32)

    w1t, w1b = W1[:D_IN], W1[D_IN:]
    w2t, w2b = W2[:H], W2[H:]
    w3t, w3b = W3[:H], W3[H:]
    b1r, g1r, bt1r = b1.reshape(1, H), g1.reshape(1, H), beta1.reshape(1, H)
    b2r, g2r, bt2r = b2.reshape(1, H), g2.reshape(1, H), beta2.reshape(1, H)
    b3r = b3.reshape(1, D_OUT)

    deg2 = _sc_degree()(row_p, ones_msg, zeros_acc)
    deg2p = deg2.reshape(NC, NP_ACC, 128)

    dinvp, zn, p1n = pl.pallas_call(
        _tc1_body,
        out_shape=[jax.ShapeDtypeStruct((NP, 128), f32),
                   jax.ShapeDtypeStruct((N, H), f32),
                   jax.ShapeDtypeStruct((N, H), f32)],
    )(deg2p, features, w1t, w1b)

    zp = zn.reshape(NP, 128)
    p1p = p1n.reshape(NP, 128)
    zlin = zp.reshape(N, H)
    dinvlin = dinvp.reshape(N, H)

    s1 = _sc_agg_scaled()(row_p, col_p, zlin, dinvlin, zeros_acc)
    v2p, p2p = pl.pallas_call(
        _tc2_body,
        out_shape=[jax.ShapeDtypeStruct((NP, 128), f32)] * 2,
    )(s1.reshape(NC, NP_ACC, 128), dinvp, zp, p1p, b1r, g1r, bt1r, w2t)

    t2 = _sc_agg()(row_p, col_p, v2p.reshape(N, H), zeros_acc)
    v3p, p3p = pl.pallas_call(
        _tc3_body,
        out_shape=[jax.ShapeDtypeStruct((NP, 128), f32),
                   jax.ShapeDtypeStruct((NP, 8 * D_OUT), f32)],
    )(t2.reshape(NC, NP_ACC, 128), dinvp, v2p, p2p, b2r, g2r, bt2r,
      w2b, w3t)

    t3 = _sc_agg()(row_p, col_p, v3p.reshape(N, H), zeros_acc)
    outp = pl.pallas_call(
        _tc4_body,
        out_shape=jax.ShapeDtypeStruct((NP, 8 * D_OUT), f32),
    )(t3.reshape(NC, NP_ACC, 128), dinvp, v3p, p3p, w3b, b3r)
    return outp.reshape(N, D_OUT)


# speedup vs baseline: 1.0727x; 1.0727x over previous
import functools

import jax
import jax.numpy as jnp
from jax import lax
from jax.experimental import pallas as pl
from jax.experimental.pallas import tpu as pltpu
from jax.experimental.pallas import tpu_sc as plsc

N = 10000
E = 320000
D_IN = 128
H = 16
D_OUT = 128

NC = 2
NS = 16
NW = NC * NS
CHUNK = 128
CHUNKS_PW = 80
EPW = CHUNK * CHUNKS_PW
E_PAD = EPW * NW
N_ACC = 10240
ROWS_PT = N_ACC // NS
NBUF = 8
GAH = 4
N_Y = N // NS

NP = N // 8
NP_ACC = N_ACC // 8


def _sc_degree_body(ei_hbm, ones_hbm, zeros_hbm, out_hbm,
                    rows_v, msg_v, acc_sh, sem):
    cid = lax.axis_index("c")
    sid = lax.axis_index("s")
    wid = cid * NS + sid
    pltpu.sync_copy(zeros_hbm.at[pl.ds(sid * ROWS_PT, ROWS_PT)],
                    acc_sh.at[pl.ds(sid * ROWS_PT, ROWS_PT)])
    pltpu.sync_copy(ei_hbm.at[wid], rows_v)
    pltpu.sync_copy(ones_hbm, msg_v)
    plsc.subcore_barrier()

    def body(cb, carry):
        for j in range(8):
            pltpu.async_copy(msg_v, acc_sh.at[rows_v.at[cb * 8 + j]], sem,
                             add=True)
        for j in range(8):
            pltpu.make_async_copy(msg_v, acc_sh.at[rows_v.at[cb * 8 + j]],
                                  sem).wait()
        return carry
    lax.fori_loop(0, CHUNKS_PW // 8, body, 0)
    plsc.subcore_barrier()
    pltpu.sync_copy(acc_sh.at[pl.ds(sid * ROWS_PT, ROWS_PT)],
                    out_hbm.at[cid, pl.ds(sid * ROWS_PT, ROWS_PT)])


@functools.cache
def _sc_degree():
    mesh = plsc.VectorSubcoreMesh(
        core_axis_name="c", subcore_axis_name="s",
        num_cores=NC, num_subcores=NS)
    return pl.kernel(
        _sc_degree_body, mesh=mesh,
        out_type=jax.ShapeDtypeStruct((NC, N_ACC, H), jnp.float32),
        scratch_types=[
            pltpu.VMEM((CHUNKS_PW, CHUNK), jnp.int32),
            pltpu.VMEM((CHUNK, H), jnp.float32),
            pltpu.VMEM_SHARED((N_ACC, H), jnp.float32),
            pltpu.SemaphoreType.DMA,
        ],
        compiler_params=pltpu.CompilerParams(use_tc_tiling_on_sc=False),
        name="sc_degree",
    )


def _agg_loop(rows_v, cols_v, msg_v, acc_sh, y_sh, gsems, ssems):
    def slot(c, j, drain, gather):
        pltpu.make_async_copy(y_sh.at[cols_v.at[c]],
                              msg_v.at[j], gsems[j]).wait()
        pltpu.async_copy(msg_v.at[j], acc_sh.at[rows_v.at[c]], ssems[j],
                         add=True)
        if gather:
            bg = (j + GAH) % NBUF
            cg = c + GAH
            if drain:
                pltpu.make_async_copy(msg_v.at[bg],
                                      acc_sh.at[rows_v.at[cg - NBUF]],
                                      ssems[bg]).wait()
            pltpu.async_copy(y_sh.at[cols_v.at[cg]], msg_v.at[bg], gsems[bg])

    for b in range(GAH):
        pltpu.async_copy(y_sh.at[cols_v.at[b]], msg_v.at[b], gsems[b])
    for j in range(NBUF):
        slot(j, j, drain=(j >= GAH), gather=True)

    def blk(cb, carry):
        for j in range(NBUF):
            slot(cb * NBUF + j, j, drain=True, gather=True)
        return carry
    lax.fori_loop(1, CHUNKS_PW // NBUF - 1, blk, 0)

    last = CHUNKS_PW - NBUF
    for j in range(NBUF):
        slot(last + j, j, drain=(j < GAH), gather=(j < GAH))
    for j in range(NBUF):
        pltpu.make_async_copy(msg_v.at[j], acc_sh.at[rows_v.at[last + j]],
                              ssems[j]).wait()


def _sc_agg_body(ei_hbm, y_hbm, zeros_hbm, out_hbm,
                 rows_v, cols_v, msg_v, acc_sh, y_sh, *sems):
    gsems, ssems = sems[:NBUF], sems[NBUF:]
    cid = lax.axis_index("c")
    sid = lax.axis_index("s")
    wid = cid * NS + sid
    pltpu.sync_copy(zeros_hbm.at[pl.ds(sid * ROWS_PT, ROWS_PT)],
                    acc_sh.at[pl.ds(sid * ROWS_PT, ROWS_PT)])
    pltpu.sync_copy(y_hbm.at[pl.ds(sid * N_Y, N_Y)],
                    y_sh.at[pl.ds(sid * N_Y, N_Y)])
    pltpu.sync_copy(ei_hbm.at[wid], rows_v)
    pltpu.sync_copy(ei_hbm.at[NW + wid], cols_v)
    plsc.subcore_barrier()
    _agg_loop(rows_v, cols_v, msg_v, acc_sh, y_sh, gsems, ssems)
    plsc.subcore_barrier()
    pltpu.sync_copy(acc_sh.at[pl.ds(sid * ROWS_PT, ROWS_PT)],
                    out_hbm.at[cid, pl.ds(sid * ROWS_PT, ROWS_PT)])


def _sc_agg_scaled_body(ei_hbm, z_hbm, dinv_hbm, zeros_hbm, out_hbm,
                        rows_v, cols_v, msg_v, acc_sh, y_sh, zb_v, db_v,
                        *sems):
    gsems, ssems = sems[:NBUF], sems[NBUF:]
    cid = lax.axis_index("c")
    sid = lax.axis_index("s")
    wid = cid * NS + sid
    pltpu.sync_copy(zeros_hbm.at[pl.ds(sid * ROWS_PT, ROWS_PT)],
                    acc_sh.at[pl.ds(sid * ROWS_PT, ROWS_PT)])
    pltpu.sync_copy(z_hbm.at[pl.ds(sid * N_Y, N_Y)], zb_v)
    pltpu.sync_copy(dinv_hbm.at[pl.ds(sid * N_Y, N_Y)], db_v)

    def scale(r, carry):
        zb_v[r] = zb_v[r] * db_v[r]
        return carry
    lax.fori_loop(0, N_Y, scale, 0)
    pltpu.sync_copy(zb_v, y_sh.at[pl.ds(sid * N_Y, N_Y)])
    pltpu.sync_copy(ei_hbm.at[wid], rows_v)
    pltpu.sync_copy(ei_hbm.at[NW + wid], cols_v)
    plsc.subcore_barrier()
    _agg_loop(rows_v, cols_v, msg_v, acc_sh, y_sh, gsems, ssems)
    plsc.subcore_barrier()
    pltpu.sync_copy(acc_sh.at[pl.ds(sid * ROWS_PT, ROWS_PT)],
                    out_hbm.at[cid, pl.ds(sid * ROWS_PT, ROWS_PT)])


_AGG_SCRATCH = [
    pltpu.VMEM((CHUNKS_PW, CHUNK), jnp.int32),
    pltpu.VMEM((CHUNKS_PW, CHUNK), jnp.int32),
    pltpu.VMEM((NBUF, CHUNK, H), jnp.float32),
    pltpu.VMEM_SHARED((N_ACC, H), jnp.float32),
    pltpu.VMEM_SHARED((N_ACC, H), jnp.float32),
]
_SEMS = [pltpu.SemaphoreType.DMA] * (2 * NBUF)


@functools.cache
def _sc_agg():
    mesh = plsc.VectorSubcoreMesh(
        core_axis_name="c", subcore_axis_name="s",
        num_cores=NC, num_subcores=NS)
    return pl.kernel(
        _sc_agg_body, mesh=mesh,
        out_type=jax.ShapeDtypeStruct((NC, N_ACC, H), jnp.float32),
        scratch_types=_AGG_SCRATCH + _SEMS,
        compiler_params=pltpu.CompilerParams(use_tc_tiling_on_sc=False),
        name="sc_agg",
    )


@functools.cache
def _sc_agg_scaled():
    mesh = plsc.VectorSubcoreMesh(
        core_axis_name="c", subcore_axis_name="s",
        num_cores=NC, num_subcores=NS)
    return pl.kernel(
        _sc_agg_scaled_body, mesh=mesh,
        out_type=jax.ShapeDtypeStruct((NC, N_ACC, H), jnp.float32),
        scratch_types=_AGG_SCRATCH + [
            pltpu.VMEM((N_Y, H), jnp.float32),
            pltpu.VMEM((N_Y, H), jnp.float32),
        ] + _SEMS,
        compiler_params=pltpu.CompilerParams(use_tc_tiling_on_sc=False),
        name="sc_agg_scaled",
    )


def _fold8(v128):
    acc = v128[:, 0:H]
    for k in range(1, 8):
        acc = acc + v128[:, k * H:(k + 1) * H]
    return acc


def _tile8(v16):
    return jnp.concatenate([v16] * 8, axis=1)


def _bn_relu_packed(h, g, beta):
    mu = _tile8(_fold8(jnp.mean(h, axis=0, keepdims=True)) / 8.0)
    ex2 = _tile8(_fold8(jnp.mean(h * h, axis=0, keepdims=True)) / 8.0)
    var = ex2 - mu * mu
    return jnp.maximum((h - mu) / jnp.sqrt(var + 1e-5) * g + beta, 0.0)


def _tc1_body(degp, x, w1t, w1b, dinvp_o, zn_o, p1n_o):
    deg = degp[0, :NP] + degp[1, :NP] + 1.0
    dinvp_o[...] = lax.rsqrt(deg)
    zn_o[...] = jnp.dot(x[...], w1b[...], preferred_element_type=jnp.float32)
    p1n_o[...] = jnp.dot(x[...], w1t[...], preferred_element_type=jnp.float32)


def _tc2_body(s1p, dinvp, zp, p1p, b1, g1, bt1, bdw2t, v2p_o, p2p_o):
    dinv = dinvp[...]
    u1 = dinv * zp[...]
    h = p1p[...] + dinv * (s1p[0, :NP] + s1p[1, :NP] + u1) + _tile8(b1[...])
    hn = _bn_relu_packed(h, _tile8(g1[...]), _tile8(bt1[...]))
    v2p_o[...] = dinv * hn
    p2p_o[...] = jnp.dot(hn, bdw2t[...], preferred_element_type=jnp.float32)


def _tc3_body(t2p, dinvp, v2p, p2p, b2, g2, bt2, bdw2b, bdw3t,
              v3p_o, p3p_o):
    dinv = dinvp[...]
    agg2 = dinv * (t2p[0, :NP] + t2p[1, :NP] + v2p[...])
    h = (p2p[...] +
         jnp.dot(agg2, bdw2b[...], preferred_element_type=jnp.float32) +
         _tile8(b2[...]))
    hn = _bn_relu_packed(h, _tile8(g2[...]), _tile8(bt2[...]))
    v3p_o[...] = dinv * hn
    p3p_o[...] = jnp.dot(hn, bdw3t[...], preferred_element_type=jnp.float32)


def _tc4_body(t3p, dinvp, v3p, p3p, bdw3b, b3, outp_o):
    agg3 = dinvp[...] * (t3p[0, :NP] + t3p[1, :NP] + v3p[...])
    b3t = jnp.concatenate([b3[...]] * 8, axis=1)
    outp_o[...] = (p3p[...] +
                   jnp.dot(agg3, bdw3b[...], preferred_element_type=jnp.float32)
                   + b3t)


def kernel(edge_index, features, W1, b1, g1, beta1, W2, b2, g2, beta2, W3, b3):
    f32 = jnp.float32
    ei3 = jnp.pad(edge_index, ((0, 0), (0, E_PAD - E)),
                  constant_values=N).reshape(2 * NW, CHUNKS_PW, CHUNK)
    zeros_acc = jnp.zeros((N_ACC, H), f32)
    ones_msg = jnp.ones((CHUNK, H), f32)

    w1t, w1b = W1[:D_IN], W1[D_IN:]
    w2t, w2b = W2[:H], W2[H:]
    w3t, w3b = W3[:H], W3[H:]
    eye8 = jnp.eye(8, dtype=f32)
    bdw2t, bdw2b = jnp.kron(eye8, w2t), jnp.kron(eye8, w2b)
    bdw3t, bdw3b = jnp.kron(eye8, w3t), jnp.kron(eye8, w3b)
    b1r, g1r, bt1r = b1.reshape(1, H), g1.reshape(1, H), beta1.reshape(1, H)
    b2r, g2r, bt2r = b2.reshape(1, H), g2.reshape(1, H), beta2.reshape(1, H)
    b3r = b3.reshape(1, D_OUT)

    deg2 = _sc_degree()(ei3, ones_msg, zeros_acc)
    deg2p = deg2.reshape(NC, NP_ACC, 128)

    dinvp, zn, p1n = pl.pallas_call(
        _tc1_body,
        out_shape=[jax.ShapeDtypeStruct((NP, 128), f32),
                   jax.ShapeDtypeStruct((N, H), f32),
                   jax.ShapeDtypeStruct((N, H), f32)],
    )(deg2p, features, w1t, w1b)

    zp = zn.reshape(NP, 128)
    p1p = p1n.reshape(NP, 128)
    zlin = zp.reshape(N, H)
    dinvlin = dinvp.reshape(N, H)

    s1 = _sc_agg_scaled()(ei3, zlin, dinvlin, zeros_acc)
    v2p, p2p = pl.pallas_call(
        _tc2_body,
        out_shape=[jax.ShapeDtypeStruct((NP, 128), f32)] * 2,
    )(s1.reshape(NC, NP_ACC, 128), dinvp, zp, p1p, b1r, g1r, bt1r, bdw2t)

    t2 = _sc_agg()(ei3, v2p.reshape(N, H), zeros_acc)
    v3p, p3p = pl.pallas_call(
        _tc3_body,
        out_shape=[jax.ShapeDtypeStruct((NP, 128), f32),
                   jax.ShapeDtypeStruct((NP, 8 * D_OUT), f32)],
    )(t2.reshape(NC, NP_ACC, 128), dinvp, v2p, p2p, b2r, g2r, bt2r,
      bdw2b, bdw3t)

    t3 = _sc_agg()(ei3, v3p.reshape(N, H), zeros_acc)
    outp = pl.pallas_call(
        _tc4_body,
        out_shape=jax.ShapeDtypeStruct((NP, 8 * D_OUT), f32),
    )(t3.reshape(NC, NP_ACC, 128), dinvp, v3p, p3p, bdw3b, b3r)
    return outp.reshape(N, D_OUT)


# speedup vs baseline: 1.1329x; 1.0561x over previous
import functools

import jax
import jax.numpy as jnp
from jax import lax
from jax.experimental import pallas as pl
from jax.experimental.pallas import tpu as pltpu
from jax.experimental.pallas import tpu_sc as plsc

N = 10000
E = 320000
D_IN = 128
H = 16
D_OUT = 128

NC = 2
NS = 16
NW = NC * NS
CHUNK = 128
CHUNKS_PW = 80
EPW = CHUNK * CHUNKS_PW
E_PAD = EPW * NW
N_ACC = 10240
ROWS_PT = N_ACC // NS
NBUF = 8
GAH = 4
N_Y = N // NS

NP = N // 8
NP_ACC = N_ACC // 8


def _sc_degree_body(ei_hbm, ones_hbm, zeros_hbm, out_hbm,
                    rows_v, msg_v, acc_sh, sem):
    cid = lax.axis_index("c")
    sid = lax.axis_index("s")
    wid = cid * NS + sid
    pltpu.sync_copy(zeros_hbm.at[pl.ds(sid * ROWS_PT, ROWS_PT)],
                    acc_sh.at[pl.ds(sid * ROWS_PT, ROWS_PT)])
    pltpu.sync_copy(ei_hbm.at[wid], rows_v)
    pltpu.sync_copy(ones_hbm, msg_v)
    plsc.subcore_barrier()

    def body(cb, carry):
        for j in range(8):
            pltpu.async_copy(msg_v, acc_sh.at[rows_v.at[cb * 8 + j]], sem,
                             add=True)
        for j in range(8):
            pltpu.make_async_copy(msg_v, acc_sh.at[rows_v.at[cb * 8 + j]],
                                  sem).wait()
        return carry
    lax.fori_loop(0, CHUNKS_PW // 8, body, 0)
    plsc.subcore_barrier()
    pltpu.sync_copy(acc_sh.at[pl.ds(sid * ROWS_PT, ROWS_PT)],
                    out_hbm.at[cid, pl.ds(sid * ROWS_PT, ROWS_PT)])


@functools.cache
def _sc_degree():
    mesh = plsc.VectorSubcoreMesh(
        core_axis_name="c", subcore_axis_name="s",
        num_cores=NC, num_subcores=NS)
    return pl.kernel(
        _sc_degree_body, mesh=mesh,
        out_type=jax.ShapeDtypeStruct((NC, N_ACC, H), jnp.float32),
        scratch_types=[
            pltpu.VMEM((CHUNKS_PW, CHUNK), jnp.int32),
            pltpu.VMEM((CHUNK, H), jnp.float32),
            pltpu.VMEM_SHARED((N_ACC, H), jnp.float32),
            pltpu.SemaphoreType.DMA,
        ],
        compiler_params=pltpu.CompilerParams(use_tc_tiling_on_sc=False),
        name="sc_degree",
    )


def _agg_loop(rows_v, cols_v, msg_v, acc_sh, y_sh, gsems, ssems):
    def slot(c, j, drain, gather):
        pltpu.make_async_copy(y_sh.at[cols_v.at[c]],
                              msg_v.at[j], gsems[j]).wait()
        pltpu.async_copy(msg_v.at[j], acc_sh.at[rows_v.at[c]], ssems[j],
                         add=True)
        if gather:
            bg = (j + GAH) % NBUF
            cg = c + GAH
            if drain:
                pltpu.make_async_copy(msg_v.at[bg],
                                      acc_sh.at[rows_v.at[cg - NBUF]],
                                      ssems[bg]).wait()
            pltpu.async_copy(y_sh.at[cols_v.at[cg]], msg_v.at[bg], gsems[bg])

    for b in range(GAH):
        pltpu.async_copy(y_sh.at[cols_v.at[b]], msg_v.at[b], gsems[b])
    for j in range(NBUF):
        slot(j, j, drain=(j >= GAH), gather=True)

    def blk(cb, carry):
        for j in range(NBUF):
            slot(cb * NBUF + j, j, drain=True, gather=True)
        return carry
    lax.fori_loop(1, CHUNKS_PW // NBUF - 1, blk, 0)

    last = CHUNKS_PW - NBUF
    for j in range(NBUF):
        slot(last + j, j, drain=(j < GAH), gather=(j < GAH))
    for j in range(NBUF):
        pltpu.make_async_copy(msg_v.at[j], acc_sh.at[rows_v.at[last + j]],
                              ssems[j]).wait()


def _sc_agg_body(ei_hbm, y_hbm, zeros_hbm, out_hbm,
                 rows_v, cols_v, msg_v, acc_sh, y_sh, *sems):
    gsems, ssems = sems[:NBUF], sems[NBUF:]
    cid = lax.axis_index("c")
    sid = lax.axis_index("s")
    wid = cid * NS + sid
    pltpu.sync_copy(zeros_hbm.at[pl.ds(sid * ROWS_PT, ROWS_PT)],
                    acc_sh.at[pl.ds(sid * ROWS_PT, ROWS_PT)])
    pltpu.sync_copy(y_hbm.at[pl.ds(sid * N_Y, N_Y)],
                    y_sh.at[pl.ds(sid * N_Y, N_Y)])
    pltpu.sync_copy(ei_hbm.at[wid], rows_v)
    pltpu.sync_copy(ei_hbm.at[NW + wid], cols_v)
    plsc.subcore_barrier()
    _agg_loop(rows_v, cols_v, msg_v, acc_sh, y_sh, gsems, ssems)
    plsc.subcore_barrier()
    pltpu.sync_copy(acc_sh.at[pl.ds(sid * ROWS_PT, ROWS_PT)],
                    out_hbm.at[cid, pl.ds(sid * ROWS_PT, ROWS_PT)])


def _sc_agg_scaled_body(ei_hbm, z_hbm, deg_hbm, zeros_hbm, out_hbm,
                        rows_v, cols_v, msg_v, acc_sh, y_sh, zb_v, da_v, db_v,
                        *sems):
    gsems, ssems = sems[:NBUF], sems[NBUF:]
    cid = lax.axis_index("c")
    sid = lax.axis_index("s")
    wid = cid * NS + sid
    pltpu.sync_copy(zeros_hbm.at[pl.ds(sid * ROWS_PT, ROWS_PT)],
                    acc_sh.at[pl.ds(sid * ROWS_PT, ROWS_PT)])
    pltpu.sync_copy(z_hbm.at[pl.ds(sid * N_Y, N_Y)], zb_v)
    pltpu.sync_copy(deg_hbm.at[0, pl.ds(sid * N_Y, N_Y)], da_v)
    pltpu.sync_copy(deg_hbm.at[1, pl.ds(sid * N_Y, N_Y)], db_v)

    def scale(r, carry):
        deg = da_v[r] + db_v[r] + 1.0
        i = plsc.bitcast(deg, jnp.int32)
        y = plsc.bitcast(0x5F3759DF - lax.shift_right_logical(i, 1),
                         jnp.float32)
        y = y * (1.5 - 0.5 * deg * y * y)
        y = y * (1.5 - 0.5 * deg * y * y)
        y = y * (1.5 - 0.5 * deg * y * y)
        zb_v[r] = zb_v[r] * y
        return carry
    lax.fori_loop(0, N_Y, scale, 0)
    pltpu.sync_copy(zb_v, y_sh.at[pl.ds(sid * N_Y, N_Y)])
    pltpu.sync_copy(ei_hbm.at[wid], rows_v)
    pltpu.sync_copy(ei_hbm.at[NW + wid], cols_v)
    plsc.subcore_barrier()
    _agg_loop(rows_v, cols_v, msg_v, acc_sh, y_sh, gsems, ssems)
    plsc.subcore_barrier()
    pltpu.sync_copy(acc_sh.at[pl.ds(sid * ROWS_PT, ROWS_PT)],
                    out_hbm.at[cid, pl.ds(sid * ROWS_PT, ROWS_PT)])


_AGG_SCRATCH = [
    pltpu.VMEM((CHUNKS_PW, CHUNK), jnp.int32),
    pltpu.VMEM((CHUNKS_PW, CHUNK), jnp.int32),
    pltpu.VMEM((NBUF, CHUNK, H), jnp.float32),
    pltpu.VMEM_SHARED((N_ACC, H), jnp.float32),
    pltpu.VMEM_SHARED((N_ACC, H), jnp.float32),
]
_SEMS = [pltpu.SemaphoreType.DMA] * (2 * NBUF)


@functools.cache
def _sc_agg():
    mesh = plsc.VectorSubcoreMesh(
        core_axis_name="c", subcore_axis_name="s",
        num_cores=NC, num_subcores=NS)
    return pl.kernel(
        _sc_agg_body, mesh=mesh,
        out_type=jax.ShapeDtypeStruct((NC, N_ACC, H), jnp.float32),
        scratch_types=_AGG_SCRATCH + _SEMS,
        compiler_params=pltpu.CompilerParams(use_tc_tiling_on_sc=False),
        name="sc_agg",
    )


@functools.cache
def _sc_agg_scaled():
    mesh = plsc.VectorSubcoreMesh(
        core_axis_name="c", subcore_axis_name="s",
        num_cores=NC, num_subcores=NS)
    return pl.kernel(
        _sc_agg_scaled_body, mesh=mesh,
        out_type=jax.ShapeDtypeStruct((NC, N_ACC, H), jnp.float32),
        scratch_types=_AGG_SCRATCH + [
            pltpu.VMEM((N_Y, H), jnp.float32),
            pltpu.VMEM((N_Y, H), jnp.float32),
            pltpu.VMEM((N_Y, H), jnp.float32),
        ] + _SEMS,
        compiler_params=pltpu.CompilerParams(use_tc_tiling_on_sc=False,
                                             needs_layout_passes=False),
        name="sc_agg_scaled",
    )


def _fold8(v128):
    acc = v128[:, 0:H]
    for k in range(1, 8):
        acc = acc + v128[:, k * H:(k + 1) * H]
    return acc


def _tile8(v16):
    return jnp.concatenate([v16] * 8, axis=1)


def _bn_relu_packed(h, g, beta):
    mu = _tile8(_fold8(jnp.mean(h, axis=0, keepdims=True)) / 8.0)
    ex2 = _tile8(_fold8(jnp.mean(h * h, axis=0, keepdims=True)) / 8.0)
    var = ex2 - mu * mu
    return jnp.maximum((h - mu) / jnp.sqrt(var + 1e-5) * g + beta, 0.0)


def _tc0_body(x, w1t, w1b, zn_o, p1n_o):
    zn_o[...] = jnp.dot(x[...], w1b[...], preferred_element_type=jnp.float32)
    p1n_o[...] = jnp.dot(x[...], w1t[...], preferred_element_type=jnp.float32)


def _dinvp(degp):
    return lax.rsqrt(degp[0, :NP] + degp[1, :NP] + 1.0)


def _tc2_body(s1p, degp, zp, p1p, b1, g1, bt1, bdw2t, v2p_o, p2p_o):
    dinv = _dinvp(degp)
    u1 = dinv * zp[...]
    h = p1p[...] + dinv * (s1p[0, :NP] + s1p[1, :NP] + u1) + _tile8(b1[...])
    hn = _bn_relu_packed(h, _tile8(g1[...]), _tile8(bt1[...]))
    v2p_o[...] = dinv * hn
    p2p_o[...] = jnp.dot(hn, bdw2t[...], preferred_element_type=jnp.float32)


def _tc3_body(t2p, degp, v2p, p2p, b2, g2, bt2, bdw2b, bdw3t,
              v3p_o, p3p_o):
    dinv = _dinvp(degp)
    agg2 = dinv * (t2p[0, :NP] + t2p[1, :NP] + v2p[...])
    h = (p2p[...] +
         jnp.dot(agg2, bdw2b[...], preferred_element_type=jnp.float32) +
         _tile8(b2[...]))
    hn = _bn_relu_packed(h, _tile8(g2[...]), _tile8(bt2[...]))
    v3p_o[...] = dinv * hn
    p3p_o[...] = jnp.dot(hn, bdw3t[...], preferred_element_type=jnp.float32)


def _tc4_body(t3p, degp, v3p, p3p, bdw3b, b3, outp_o):
    agg3 = _dinvp(degp) * (t3p[0, :NP] + t3p[1, :NP] + v3p[...])
    b3t = jnp.concatenate([b3[...]] * 8, axis=1)
    outp_o[...] = (p3p[...] +
                   jnp.dot(agg3, bdw3b[...], preferred_element_type=jnp.float32)
                   + b3t)


def kernel(edge_index, features, W1, b1, g1, beta1, W2, b2, g2, beta2, W3, b3):
    f32 = jnp.float32
    ei3 = jnp.pad(edge_index, ((0, 0), (0, E_PAD - E)),
                  constant_values=N).reshape(2 * NW, CHUNKS_PW, CHUNK)
    zeros_acc = jnp.zeros((N_ACC, H), f32)
    ones_msg = jnp.ones((CHUNK, H), f32)

    w1t, w1b = W1[:D_IN], W1[D_IN:]
    w2t, w2b = W2[:H], W2[H:]
    w3t, w3b = W3[:H], W3[H:]
    eye8 = jnp.eye(8, dtype=f32)
    bdw2t, bdw2b = jnp.kron(eye8, w2t), jnp.kron(eye8, w2b)
    bdw3t, bdw3b = jnp.kron(eye8, w3t), jnp.kron(eye8, w3b)
    b1r, g1r, bt1r = b1.reshape(1, H), g1.reshape(1, H), beta1.reshape(1, H)
    b2r, g2r, bt2r = b2.reshape(1, H), g2.reshape(1, H), beta2.reshape(1, H)
    b3r = b3.reshape(1, D_OUT)

    deg2 = _sc_degree()(ei3, ones_msg, zeros_acc)
    deg2p = deg2.reshape(NC, NP_ACC, 128)

    zn, p1n = pl.pallas_call(
        _tc0_body,
        out_shape=[jax.ShapeDtypeStruct((N, H), f32)] * 2,
    )(features, w1t, w1b)

    zp = zn.reshape(NP, 128)
    p1p = p1n.reshape(NP, 128)
    zlin = zp.reshape(N, H)

    s1 = _sc_agg_scaled()(ei3, zlin, deg2, zeros_acc)
    v2p, p2p = pl.pallas_call(
        _tc2_body,
        out_shape=[jax.ShapeDtypeStruct((NP, 128), f32)] * 2,
    )(s1.reshape(NC, NP_ACC, 128), deg2p, zp, p1p, b1r, g1r, bt1r, bdw2t)

    t2 = _sc_agg()(ei3, v2p.reshape(N, H), zeros_acc)
    v3p, p3p = pl.pallas_call(
        _tc3_body,
        out_shape=[jax.ShapeDtypeStruct((NP, 128), f32),
                   jax.ShapeDtypeStruct((NP, 8 * D_OUT), f32)],
    )(t2.reshape(NC, NP_ACC, 128), deg2p, v2p, p2p, b2r, g2r, bt2r,
      bdw2b, bdw3t)

    t3 = _sc_agg()(ei3, v3p.reshape(N, H), zeros_acc)
    outp = pl.pallas_call(
        _tc4_body,
        out_shape=jax.ShapeDtypeStruct((NP, 8 * D_OUT), f32),
    )(t3.reshape(NC, NP_ACC, 128), deg2p, v3p, p3p, bdw3b, b3r)
    return outp.reshape(N, D_OUT)
